# baseline (speedup 1.0000x reference)
import functools

import jax
import jax.numpy as jnp
from jax import lax
from jax.experimental import pallas as pl
from jax.experimental.pallas import tpu as pltpu
from jax.experimental.pallas import tpu_sc as plsc

B = 16384
D = 16
L = 16
NC = 2
NS = 16
NW = NC * NS
BPW = B // NW
NCH = BPW // 128
UN = 1000000
IN = 100000
UNG = UN // L
ING = IN // L

_LOG1P_COEF = (
    9.09903358e-08, 9.99991449e-01, -4.99801099e-01, 3.31333659e-01,
    -2.39189722e-01, 1.64781887e-01, -9.23123095e-02, 3.44179115e-02,
    -6.07475245e-03,
)


def _softplus(x):
    t = jnp.exp(-jnp.abs(x))
    p = jnp.full((L,), _LOG1P_COEF[-1], jnp.float32)
    for c in _LOG1P_COEF[-2::-1]:
        p = p * t + c
    return jnp.maximum(x, 0.0) + p


@functools.partial(
    pl.kernel,
    out_type=jax.ShapeDtypeStruct((B,), jnp.float32),
    mesh=plsc.VectorSubcoreMesh(core_axis_name="c", subcore_axis_name="s"),
    compiler_params=pltpu.CompilerParams(
        needs_layout_passes=False, use_tc_tiling_on_sc=False),
    scratch_types=[
        pltpu.VMEM((NCH, 128), jnp.int32),
        pltpu.VMEM((NCH, 128), jnp.int32),
        pltpu.VMEM((D * BPW,), jnp.int32),
        pltpu.VMEM((D * BPW,), jnp.int32),
        pltpu.VMEM((BPW,), jnp.int32),
        pltpu.VMEM((BPW,), jnp.int32),
        pltpu.VMEM((BPW, L), jnp.float32),
        pltpu.VMEM((BPW, L), jnp.float32),
        pltpu.VMEM((BPW, L), jnp.float32),
        pltpu.VMEM((BPW, L), jnp.float32),
        pltpu.VMEM((BPW,), jnp.float32),
        pltpu.VMEM((BPW,), jnp.float32),
        pltpu.VMEM((BPW,), jnp.float32),
        pltpu.SemaphoreType.DMA,
        pltpu.SemaphoreType.DMA,
        pltpu.SemaphoreType.DMA,
    ],
)
def _mirt_sc(user_hbm, item_hbm, theta_hbm, a_hbm, b_hbm, out_hbm,
             uidx_v, iidx_v, trow_v, arow_v, tcol_v, acol_v,
             tb0, tb1, ab0, ab1, b_v, acc_v, out_v, sem0, sem1, semb):
    wid = lax.axis_index("s") * NC + lax.axis_index("c")

    pltpu.sync_copy(user_hbm.at[pl.ds(wid * NCH, NCH)], uidx_v)
    pltpu.sync_copy(item_hbm.at[pl.ds(wid * NCH, NCH)], iidx_v)

    bcopies = [
        pltpu.async_copy(b_hbm.at[iidx_v.at[j]], b_v.at[pl.ds(j * 128, 128)],
                         semb)
        for j in range(NCH)
    ]

    for j in range(NCH):
        for v in range(8):
            sl = pl.ds(j * 128 + v * L, L)
            src = pl.ds(v * L, L)
            uvec = uidx_v[j, src]
            ivec = iidx_v[j, src]
            tcol_v[sl] = lax.bitwise_and(uvec, L - 1)
            acol_v[sl] = lax.bitwise_and(ivec, L - 1)

    def row_body(k, carry):
        for j in range(NCH):
            for v in range(8):
                sl = pl.ds(j * 128 + v * L, L)
                src = pl.ds(v * L, L)
                uvec = uidx_v[j, src]
                ivec = iidx_v[j, src]
                dst = pl.ds(k * BPW + j * 128 + v * L, L)
                trow_v[dst] = lax.shift_right_logical(uvec, 4) + k * UNG
                arow_v[dst] = lax.shift_right_logical(ivec, 4) + k * ING
        return carry

    lax.fori_loop(0, D, row_body, 0)

    def fire(k, tbuf, abuf, sem):
        sl = pl.ds(k * BPW, BPW)
        return (pltpu.async_copy(theta_hbm.at[trow_v.at[sl]], tbuf, sem),
                pltpu.async_copy(a_hbm.at[arow_v.at[sl]], abuf, sem))

    tbufs = (tb0, tb1)
    abufs = (ab0, ab1)
    sems = (sem0, sem1)
    inflight = fire(0, tb0, ab0, sem0)

    erow = lax.iota(jnp.int32, L)

    for k in range(D):
        nxt = None
        if k + 1 < D:
            nxt = fire(k + 1, tbufs[(k + 1) % 2], abufs[(k + 1) % 2],
                       sems[(k + 1) % 2])
        for c in inflight:
            c.wait()
        inflight = nxt
        tbuf = tbufs[k % 2]
        abuf = abufs[k % 2]

        def grp(g, carry, k=k, tbuf=tbuf, abuf=abuf):
            sl = pl.ds(g * L, L)
            rows = erow + g * L
            vt = plsc.load_gather(tbuf, [rows, tcol_v[sl]])
            va = plsc.load_gather(abuf, [rows, acol_v[sl]])
            part = _softplus(va) * vt
            if k == 0:
                acc_v[sl] = part
            else:
                acc_v[sl] = acc_v[sl] + part
            return carry

        lax.fori_loop(0, BPW // L, grp, 0)

    for c in bcopies:
        c.wait()

    def fin(g, carry):
        sl = pl.ds(g * L, L)
        out_v[sl] = 1.0 / (1.0 + jnp.exp(b_v[sl] - acc_v[sl]))
        return carry

    lax.fori_loop(0, BPW // L, fin, 0)
    pltpu.sync_copy(out_v, out_hbm.at[pl.ds(wid * BPW, BPW)])


def kernel(user, item, theta_table, a_table, b_table):
    u2 = user.astype(jnp.int32).reshape(NW * NCH, 128)
    i2 = item.astype(jnp.int32).reshape(NW * NCH, 128)
    th_view = theta_table.T.reshape(UN * D // L, L)
    a_view = a_table.T.reshape(IN * D // L, L)
    b1 = b_table.reshape((IN,))
    return _mirt_sc(u2, i2, th_view, a_view, b1)


# speedup vs baseline: 5.3957x; 5.3957x over previous
import functools

import jax
import jax.numpy as jnp
from jax import lax
from jax.experimental import pallas as pl
from jax.experimental.pallas import tpu as pltpu
from jax.experimental.pallas import tpu_sc as plsc

B = 16384
D = 16
L = 16
NC = 2
NS = 16
NW = NC * NS
BPW = B // NW
NCH = BPW // 128
UN = 1000000
IN = 100000

WW = 8192
UFULL = UN // WW
UA_C0 = UFULL * WW
UA_CW = 512
UB_C0 = UA_C0 + UA_CW
AFULL = IN // WW
AA_C0 = AFULL * WW
AA_CW = 1664
AB_C0 = AA_C0 + AA_CW

_LOG1P_COEF = (
    9.09903358e-08, 9.99991449e-01, -4.99801099e-01, 3.31333659e-01,
    -2.39189722e-01, 1.64781887e-01, -9.23123095e-02, 3.44179115e-02,
    -6.07475245e-03,
)


def _softplus(x):
    t = jnp.exp(-jnp.abs(x))
    p = jnp.full((L,), _LOG1P_COEF[-1], jnp.float32)
    for c in _LOG1P_COEF[-2::-1]:
        p = p * t + c
    return jnp.maximum(x, 0.0) + p


@functools.partial(
    pl.kernel,
    out_type=(jax.ShapeDtypeStruct((D * UN,), jnp.float32),
              jax.ShapeDtypeStruct((D * IN,), jnp.float32)),
    mesh=plsc.VectorSubcoreMesh(core_axis_name="c", subcore_axis_name="s"),
    compiler_params=pltpu.CompilerParams(
        needs_layout_passes=False, use_tc_tiling_on_sc=True),
    scratch_types=[
        pltpu.VMEM((8, WW), jnp.float32),
        pltpu.VMEM((8 * WW,), jnp.float32),
        pltpu.SemaphoreType.DMA,
    ],
)
def _detile(th_hbm, a_hbm, tht_hbm, at_hbm, tout_hbm, aout_hbm,
            buf_v, row_v, sem):
    wid = lax.axis_index("s") * NC + lax.axis_index("c")

    def window(src, out, n, tr, c0, cw, out_c0=None):
        if out_c0 is None:
            out_c0 = c0
        pltpu.sync_copy(src.at[pl.ds(tr * 8, 8), pl.ds(c0, cw)],
                        buf_v.at[:, pl.ds(0, cw)])

        def mv(xi, carry):
            for v in range(8):
                sl = pl.ds(xi * 128 + v * L, L)
                for kk in range(8):
                    dsl = pl.ds(kk * WW + xi * 128 + v * L, L)
                    row_v[dsl] = buf_v[kk, sl]
            return carry

        lax.fori_loop(0, cw // 128, mv, 0)
        for kk in range(8):
            pltpu.sync_copy(
                row_v.at[pl.ds(kk * WW, cw)],
                out.at[pl.ds((tr * 8 + kk) * n + out_c0, cw)])

    def tjob(j, carry):
        tr = j // UFULL
        w = j % UFULL

        @pl.when(j % NW == wid)
        def _():
            window(th_hbm, tout_hbm, UN, tr, w * WW, WW)
        return carry

    lax.fori_loop(0, 2 * UFULL, tjob, 0)

    def ajob(j, carry):
        tr = j // AFULL
        w = j % AFULL

        @pl.when(j % NW == wid)
        def _():
            window(a_hbm, aout_hbm, IN, tr, w * WW, WW)
        return carry

    lax.fori_loop(0, 2 * AFULL, ajob, 0)

    @pl.when(wid == 0)
    def _():
        window(th_hbm, tout_hbm, UN, 0, UA_C0, UA_CW)

    @pl.when(wid == 1)
    def _():
        window(th_hbm, tout_hbm, UN, 1, UA_C0, UA_CW)

    @pl.when(wid == 2)
    def _():
        window(a_hbm, aout_hbm, IN, 0, AA_C0, AA_CW)

    @pl.when(wid == 3)
    def _():
        window(a_hbm, aout_hbm, IN, 1, AA_C0, AA_CW)

    @pl.when(wid == 4)
    def _():
        window(tht_hbm, tout_hbm, UN, 0, 0, 128, out_c0=UN - 128)

    @pl.when(wid == 5)
    def _():
        window(tht_hbm, tout_hbm, UN, 1, 0, 128, out_c0=UN - 128)

    @pl.when(wid == 6)
    def _():
        window(at_hbm, aout_hbm, IN, 0, 0, 128, out_c0=IN - 128)

    @pl.when(wid == 7)
    def _():
        window(at_hbm, aout_hbm, IN, 1, 0, 128, out_c0=IN - 128)


@functools.partial(
    pl.kernel,
    out_type=jax.ShapeDtypeStruct((B,), jnp.float32),
    mesh=plsc.VectorSubcoreMesh(core_axis_name="c", subcore_axis_name="s"),
    compiler_params=pltpu.CompilerParams(
        needs_layout_passes=False, use_tc_tiling_on_sc=False),
    scratch_types=[
        pltpu.VMEM((NCH, 128), jnp.int32),
        pltpu.VMEM((NCH, 128), jnp.int32),
        pltpu.VMEM((D * BPW,), jnp.int32),
        pltpu.VMEM((D * BPW,), jnp.int32),
        pltpu.VMEM((D * BPW,), jnp.float32),
        pltpu.VMEM((D * BPW,), jnp.float32),
        pltpu.VMEM((BPW,), jnp.float32),
        pltpu.VMEM((BPW,), jnp.float32),
        pltpu.SemaphoreType.DMA,
    ],
)
def _mirt_sc(user_hbm, item_hbm, theta_hbm, a_hbm, b_hbm, out_hbm,
             uidx_v, iidx_v, thidx_v, aidx_v, th_v, a_v, b_v, out_v, sem):
    wid = lax.axis_index("s") * NC + lax.axis_index("c")

    pltpu.sync_copy(user_hbm.at[pl.ds(wid * NCH, NCH)], uidx_v)
    pltpu.sync_copy(item_hbm.at[pl.ds(wid * NCH, NCH)], iidx_v)

    def idx_body(k, carry):
        ush = k * UN
        ish = k * IN
        for j in range(NCH):
            for v in range(8):
                src = pl.ds(v * L, L)
                dst = pl.ds(k * BPW + j * 128 + v * L, L)
                thidx_v[dst] = uidx_v[j, src] + ush
                aidx_v[dst] = iidx_v[j, src] + ish
        return carry

    lax.fori_loop(0, D, idx_body, 0)

    copies = []
    for k in range(D):
        sl = pl.ds(k * BPW, BPW)
        copies.append(pltpu.async_copy(
            theta_hbm.at[thidx_v.at[sl]], th_v.at[sl], sem))
        copies.append(pltpu.async_copy(
            a_hbm.at[aidx_v.at[sl]], a_v.at[sl], sem))
    for j in range(NCH):
        sl = pl.ds(j * 128, 128)
        copies.append(pltpu.async_copy(b_hbm.at[iidx_v.at[j]], b_v.at[sl], sem))
    for c in copies:
        c.wait()

    def group_body(g, carry):
        rows = lax.iota(jnp.int32, L) + g * L
        acc = jnp.zeros((L,), jnp.float32)
        for k in range(D):
            sl = pl.ds(k * BPW + g * L, L)
            acc = acc + _softplus(a_v[sl]) * th_v[sl]
        vb = plsc.load_gather(b_v, [rows])
        res = 1.0 / (1.0 + jnp.exp(vb - acc))
        plsc.store_scatter(out_v, [rows], res)
        return carry

    lax.fori_loop(0, BPW // L, group_body, 0)
    pltpu.sync_copy(out_v, out_hbm.at[pl.ds(wid * BPW, BPW)])


def kernel(user, item, theta_table, a_table, b_table):
    u2 = user.astype(jnp.int32).reshape(NW * NCH, 128)
    i2 = item.astype(jnp.int32).reshape(NW * NCH, 128)
    th_lin, a_lin = _detile(theta_table.T, a_table.T,
                            theta_table[UN - 128:].T, a_table[IN - 128:].T)
    b1 = b_table.reshape((IN,))
    return _mirt_sc(u2, i2, th_lin, a_lin, b1)


# speedup vs baseline: 5.4526x; 1.0105x over previous
import functools

import jax
import jax.numpy as jnp
from jax import lax
from jax.experimental import pallas as pl
from jax.experimental.pallas import tpu as pltpu
from jax.experimental.pallas import tpu_sc as plsc

B = 16384
D = 16
L = 16
NC = 2
NS = 16
NW = NC * NS
BPW = B // NW
NCH = BPW // 128
UN = 1000000
IN = 100000

WW = 4096
UFULL = UN // WW
UA_C0 = UFULL * WW
UA_CW = 512
AFULL = IN // WW
AA_C0 = AFULL * WW
AA_CW = 1664
NJT_H = 256
NJA_H = 32

_LOG1P_COEF = (
    9.09903358e-08, 9.99991449e-01, -4.99801099e-01, 3.31333659e-01,
    -2.39189722e-01, 1.64781887e-01, -9.23123095e-02, 3.44179115e-02,
    -6.07475245e-03,
)


def _softplus(x):
    t = jnp.exp(-jnp.abs(x))
    p = jnp.full((L,), _LOG1P_COEF[-1], jnp.float32)
    for c in _LOG1P_COEF[-2::-1]:
        p = p * t + c
    return jnp.maximum(x, 0.0) + p


@functools.partial(
    pl.kernel,
    out_type=(jax.ShapeDtypeStruct((D * UN,), jnp.float32),
              jax.ShapeDtypeStruct((D * IN,), jnp.float32)),
    mesh=plsc.VectorSubcoreMesh(core_axis_name="c", subcore_axis_name="s"),
    compiler_params=pltpu.CompilerParams(
        needs_layout_passes=False, use_tc_tiling_on_sc=True),
    scratch_types=[
        pltpu.VMEM((8, WW), jnp.float32),
        pltpu.VMEM((8 * WW,), jnp.float32),
        pltpu.VMEM((8 * WW,), jnp.float32),
        pltpu.SemaphoreType.DMA,
        pltpu.SemaphoreType.DMA,
    ],
)
def _detile(th_hbm, a_hbm, tht_hbm, at_hbm, tout_hbm, aout_hbm,
            buf_v, row0_v, row1_v, sem0, sem1, ):
    wid = lax.axis_index("s") * NC + lax.axis_index("c")

    def detile_to(src, n, tr, c0, cw, row_v):
        pltpu.sync_copy(src.at[pl.ds(tr * 8, 8), pl.ds(c0, cw)],
                        buf_v.at[:, pl.ds(0, cw)])

        def mv(xi, carry):
            for v in range(8):
                sl = pl.ds(xi * 128 + v * L, L)
                for kk in range(8):
                    dsl = pl.ds(kk * WW + xi * 128 + v * L, L)
                    row_v[dsl] = buf_v[kk, sl]
            return carry

        lax.fori_loop(0, cw // 128, mv, 0)

    def fire_writes(out, n, tr, out_c0, cw, row_v, sem):
        for kk in range(8):
            pltpu.async_copy(
                row_v.at[pl.ds(kk * WW, cw)],
                out.at[pl.ds((tr * 8 + kk) * n + out_c0, cw)], sem)

    def drain_writes(out, cw, row_v, sem):
        for kk in range(8):
            pltpu.make_async_copy(
                row_v.at[pl.ds(kk * WW, cw)],
                out.at[pl.ds(kk * cw, cw)], sem).wait()

    def window(src, out, n, tr, c0, cw, out_c0=None):
        if out_c0 is None:
            out_c0 = c0
        detile_to(src, n, tr, c0, cw, row0_v)
        fire_writes(out, n, tr, out_c0, cw, row0_v, sem0)
        drain_writes(out, cw, row0_v, sem0)

    def jmap(j, half, real):
        tr = j // half
        w = lax.rem(j % half, real)
        return tr, w

    def tpair(i, carry):
        j0 = wid + (2 * i) * NW
        j1 = wid + (2 * i + 1) * NW

        @pl.when(i > 0)
        def _():
            drain_writes(tout_hbm, WW, row0_v, sem0)
        tr0, w0 = jmap(j0, NJT_H, UFULL)
        detile_to(th_hbm, UN, tr0, w0 * WW, WW, row0_v)
        fire_writes(tout_hbm, UN, tr0, w0 * WW, WW, row0_v, sem0)

        @pl.when(i > 0)
        def _():
            drain_writes(tout_hbm, WW, row1_v, sem1)
        tr1, w1 = jmap(j1, NJT_H, UFULL)
        detile_to(th_hbm, UN, tr1, w1 * WW, WW, row1_v)
        fire_writes(tout_hbm, UN, tr1, w1 * WW, WW, row1_v, sem1)
        return carry

    lax.fori_loop(0, NJT_H * 2 // (2 * NW), tpair, 0)
    drain_writes(tout_hbm, WW, row0_v, sem0)
    drain_writes(tout_hbm, WW, row1_v, sem1)

    ja0 = wid
    ja1 = wid + NW
    tra0, wa0 = jmap(ja0, NJA_H, AFULL)
    detile_to(a_hbm, IN, tra0, wa0 * WW, WW, row0_v)
    fire_writes(aout_hbm, IN, tra0, wa0 * WW, WW, row0_v, sem0)
    tra1, wa1 = jmap(ja1, NJA_H, AFULL)
    detile_to(a_hbm, IN, tra1, wa1 * WW, WW, row1_v)
    fire_writes(aout_hbm, IN, tra1, wa1 * WW, WW, row1_v, sem1)
    drain_writes(aout_hbm, WW, row0_v, sem0)
    drain_writes(aout_hbm, WW, row1_v, sem1)

    @pl.when(wid == 0)
    def _():
        window(th_hbm, tout_hbm, UN, 0, UA_C0, UA_CW)

    @pl.when(wid == 1)
    def _():
        window(th_hbm, tout_hbm, UN, 1, UA_C0, UA_CW)

    @pl.when(wid == 2)
    def _():
        window(a_hbm, aout_hbm, IN, 0, AA_C0, AA_CW)

    @pl.when(wid == 3)
    def _():
        window(a_hbm, aout_hbm, IN, 1, AA_C0, AA_CW)

    @pl.when(wid == 4)
    def _():
        window(tht_hbm, tout_hbm, UN, 0, 0, 128, out_c0=UN - 128)

    @pl.when(wid == 5)
    def _():
        window(tht_hbm, tout_hbm, UN, 1, 0, 128, out_c0=UN - 128)

    @pl.when(wid == 6)
    def _():
        window(at_hbm, aout_hbm, IN, 0, 0, 128, out_c0=IN - 128)

    @pl.when(wid == 7)
    def _():
        window(at_hbm, aout_hbm, IN, 1, 0, 128, out_c0=IN - 128)


@functools.partial(
    pl.kernel,
    out_type=jax.ShapeDtypeStruct((B,), jnp.float32),
    mesh=plsc.VectorSubcoreMesh(core_axis_name="c", subcore_axis_name="s"),
    compiler_params=pltpu.CompilerParams(
        needs_layout_passes=False, use_tc_tiling_on_sc=False),
    scratch_types=[
        pltpu.VMEM((NCH, 128), jnp.int32),
        pltpu.VMEM((NCH, 128), jnp.int32),
        pltpu.VMEM((D * BPW,), jnp.int32),
        pltpu.VMEM((D * BPW,), jnp.int32),
        pltpu.VMEM((D * BPW,), jnp.float32),
        pltpu.VMEM((D * BPW,), jnp.float32),
        pltpu.VMEM((BPW,), jnp.float32),
        pltpu.VMEM((BPW,), jnp.float32),
        pltpu.SemaphoreType.DMA,
    ],
)
def _mirt_sc(user_hbm, item_hbm, theta_hbm, a_hbm, b_hbm, out_hbm,
             uidx_v, iidx_v, thidx_v, aidx_v, th_v, a_v, b_v, out_v, sem):
    wid = lax.axis_index("s") * NC + lax.axis_index("c")

    pltpu.sync_copy(user_hbm.at[pl.ds(wid * NCH, NCH)], uidx_v)
    pltpu.sync_copy(item_hbm.at[pl.ds(wid * NCH, NCH)], iidx_v)

    def idx_body(k, carry):
        ush = k * UN
        ish = k * IN
        for j in range(NCH):
            for v in range(8):
                src = pl.ds(v * L, L)
                dst = pl.ds(k * BPW + j * 128 + v * L, L)
                thidx_v[dst] = uidx_v[j, src] + ush
                aidx_v[dst] = iidx_v[j, src] + ish
        return carry

    lax.fori_loop(0, D, idx_body, 0)

    copies = []
    for k in range(D):
        sl = pl.ds(k * BPW, BPW)
        copies.append(pltpu.async_copy(
            theta_hbm.at[thidx_v.at[sl]], th_v.at[sl], sem))
        copies.append(pltpu.async_copy(
            a_hbm.at[aidx_v.at[sl]], a_v.at[sl], sem))
    for j in range(NCH):
        sl = pl.ds(j * 128, 128)
        copies.append(pltpu.async_copy(b_hbm.at[iidx_v.at[j]], b_v.at[sl], sem))
    for c in copies:
        c.wait()

    def group_body(g, carry):
        rows = lax.iota(jnp.int32, L) + g * L
        acc = jnp.zeros((L,), jnp.float32)
        for k in range(D):
            sl = pl.ds(k * BPW + g * L, L)
            acc = acc + _softplus(a_v[sl]) * th_v[sl]
        vb = plsc.load_gather(b_v, [rows])
        res = 1.0 / (1.0 + jnp.exp(vb - acc))
        plsc.store_scatter(out_v, [rows], res)
        return carry

    lax.fori_loop(0, BPW // L, group_body, 0)
    pltpu.sync_copy(out_v, out_hbm.at[pl.ds(wid * BPW, BPW)])


def kernel(user, item, theta_table, a_table, b_table):
    u2 = user.astype(jnp.int32).reshape(NW * NCH, 128)
    i2 = item.astype(jnp.int32).reshape(NW * NCH, 128)
    th_lin, a_lin = _detile(theta_table.T, a_table.T,
                            theta_table[UN - 128:].T, a_table[IN - 128:].T)
    b1 = b_table.reshape((IN,))
    return _mirt_sc(u2, i2, th_lin, a_lin, b1)


# speedup vs baseline: 7.3180x; 1.3421x over previous
import functools

import jax
import jax.numpy as jnp
from jax import lax
from jax.experimental import pallas as pl
from jax.experimental.pallas import tpu as pltpu
from jax.experimental.pallas import tpu_sc as plsc

B = 16384
D = 16
L = 16
NC = 2
NS = 16
NW = NC * NS
BPW = B // NW
NCH = BPW // 128
UN = 1000000
IN = 100000

WW = 2048
UFULL = UN // WW
UA_C0 = UFULL * WW
UA_CW = 512
AFULL = IN // WW
AA_C0 = AFULL * WW
AA_CW = 1664
NJT_H = 512
NJA_H = 64

_LOG1P_COEF = (
    9.09903358e-08, 9.99991449e-01, -4.99801099e-01, 3.31333659e-01,
    -2.39189722e-01, 1.64781887e-01, -9.23123095e-02, 3.44179115e-02,
    -6.07475245e-03,
)


def _softplus(x):
    t = jnp.exp(-jnp.abs(x))
    p = jnp.full((L,), _LOG1P_COEF[-1], jnp.float32)
    for c in _LOG1P_COEF[-2::-1]:
        p = p * t + c
    return jnp.maximum(x, 0.0) + p


@functools.partial(
    pl.kernel,
    out_type=(jax.ShapeDtypeStruct((D * UN,), jnp.float32),
              jax.ShapeDtypeStruct((D * IN,), jnp.float32)),
    mesh=plsc.VectorSubcoreMesh(core_axis_name="c", subcore_axis_name="s"),
    compiler_params=pltpu.CompilerParams(
        needs_layout_passes=False, use_tc_tiling_on_sc=True),
    scratch_types=[
        pltpu.VMEM((8, WW), jnp.float32),
        pltpu.VMEM((8 * WW,), jnp.float32),
        pltpu.VMEM((8 * WW,), jnp.float32),
        pltpu.SemaphoreType.DMA,
        pltpu.SemaphoreType.DMA,
    ],
)
def _detile(th_hbm, a_hbm, tht_hbm, at_hbm, tout_hbm, aout_hbm,
            buf_v, row0_v, row1_v, sem0, sem1, ):
    wid = lax.axis_index("s") * NC + lax.axis_index("c")

    def detile_to(src, n, tr, c0, cw, row_v, unroll=False):
        pltpu.sync_copy(src.at[pl.ds(tr * 8, 8), pl.ds(c0, cw)],
                        buf_v.at[:, pl.ds(0, cw)])

        def step(xi):
            for v in range(8):
                sl = pl.ds(xi * 128 + v * L, L)
                for kk in range(8):
                    dsl = pl.ds(kk * WW + xi * 128 + v * L, L)
                    row_v[dsl] = buf_v[kk, sl]

        if unroll:
            for xi in range(cw // 128):
                step(xi)
        else:
            def mv(xi, carry):
                step(xi)
                return carry

            lax.fori_loop(0, cw // 128, mv, 0)

    def fire_writes(out, n, tr, out_c0, cw, row_v, sem):
        for kk in range(8):
            pltpu.async_copy(
                row_v.at[pl.ds(kk * WW, cw)],
                out.at[pl.ds((tr * 8 + kk) * n + out_c0, cw)], sem)

    def drain_writes(out, cw, row_v, sem):
        for kk in range(8):
            pltpu.make_async_copy(
                row_v.at[pl.ds(kk * WW, cw)],
                out.at[pl.ds(kk * cw, cw)], sem).wait()

    def window(src, out, n, tr, c0, cw, out_c0=None):
        if out_c0 is None:
            out_c0 = c0
        detile_to(src, n, tr, c0, cw, row0_v)
        fire_writes(out, n, tr, out_c0, cw, row0_v, sem0)
        drain_writes(out, cw, row0_v, sem0)

    def jmap(j, half, real):
        tr = j // half
        w = lax.rem(j % half, real)
        return tr, w

    def tpair(i, carry):
        j0 = wid + (2 * i) * NW
        j1 = wid + (2 * i + 1) * NW

        @pl.when(i > 0)
        def _():
            drain_writes(tout_hbm, WW, row0_v, sem0)
        tr0, w0 = jmap(j0, NJT_H, UFULL)
        detile_to(th_hbm, UN, tr0, w0 * WW, WW, row0_v, unroll=True)
        fire_writes(tout_hbm, UN, tr0, w0 * WW, WW, row0_v, sem0)

        @pl.when(i > 0)
        def _():
            drain_writes(tout_hbm, WW, row1_v, sem1)
        tr1, w1 = jmap(j1, NJT_H, UFULL)
        detile_to(th_hbm, UN, tr1, w1 * WW, WW, row1_v, unroll=True)
        fire_writes(tout_hbm, UN, tr1, w1 * WW, WW, row1_v, sem1)
        return carry

    lax.fori_loop(0, NJT_H * 2 // (2 * NW), tpair, 0)
    drain_writes(tout_hbm, WW, row0_v, sem0)
    drain_writes(tout_hbm, WW, row1_v, sem1)

    def apair(i, carry):
        j0 = wid + (2 * i) * NW
        j1 = wid + (2 * i + 1) * NW

        @pl.when(i > 0)
        def _():
            drain_writes(aout_hbm, WW, row0_v, sem0)
        tr0, w0 = jmap(j0, NJA_H, AFULL)
        detile_to(a_hbm, IN, tr0, w0 * WW, WW, row0_v, unroll=True)
        fire_writes(aout_hbm, IN, tr0, w0 * WW, WW, row0_v, sem0)

        @pl.when(i > 0)
        def _():
            drain_writes(aout_hbm, WW, row1_v, sem1)
        tr1, w1 = jmap(j1, NJA_H, AFULL)
        detile_to(a_hbm, IN, tr1, w1 * WW, WW, row1_v, unroll=True)
        fire_writes(aout_hbm, IN, tr1, w1 * WW, WW, row1_v, sem1)
        return carry

    lax.fori_loop(0, NJA_H * 2 // (2 * NW), apair, 0)
    drain_writes(aout_hbm, WW, row0_v, sem0)
    drain_writes(aout_hbm, WW, row1_v, sem1)

    @pl.when(wid == 0)
    def _():
        window(th_hbm, tout_hbm, UN, 0, UA_C0, UA_CW)

    @pl.when(wid == 1)
    def _():
        window(th_hbm, tout_hbm, UN, 1, UA_C0, UA_CW)

    @pl.when(wid == 2)
    def _():
        window(a_hbm, aout_hbm, IN, 0, AA_C0, AA_CW)

    @pl.when(wid == 3)
    def _():
        window(a_hbm, aout_hbm, IN, 1, AA_C0, AA_CW)

    @pl.when(wid == 4)
    def _():
        window(tht_hbm, tout_hbm, UN, 0, 0, 128, out_c0=UN - 128)

    @pl.when(wid == 5)
    def _():
        window(tht_hbm, tout_hbm, UN, 1, 0, 128, out_c0=UN - 128)

    @pl.when(wid == 6)
    def _():
        window(at_hbm, aout_hbm, IN, 0, 0, 128, out_c0=IN - 128)

    @pl.when(wid == 7)
    def _():
        window(at_hbm, aout_hbm, IN, 1, 0, 128, out_c0=IN - 128)


@functools.partial(
    pl.kernel,
    out_type=jax.ShapeDtypeStruct((B,), jnp.float32),
    mesh=plsc.VectorSubcoreMesh(core_axis_name="c", subcore_axis_name="s"),
    compiler_params=pltpu.CompilerParams(
        needs_layout_passes=False, use_tc_tiling_on_sc=False),
    scratch_types=[
        pltpu.VMEM((NCH, 128), jnp.int32),
        pltpu.VMEM((NCH, 128), jnp.int32),
        pltpu.VMEM((D * BPW,), jnp.int32),
        pltpu.VMEM((D * BPW,), jnp.int32),
        pltpu.VMEM((D * BPW,), jnp.float32),
        pltpu.VMEM((D * BPW,), jnp.float32),
        pltpu.VMEM((BPW,), jnp.float32),
        pltpu.VMEM((BPW,), jnp.float32),
        pltpu.SemaphoreType.DMA,
    ],
)
def _mirt_sc(user_hbm, item_hbm, theta_hbm, a_hbm, b_hbm, out_hbm,
             uidx_v, iidx_v, thidx_v, aidx_v, th_v, a_v, b_v, out_v, sem):
    wid = lax.axis_index("s") * NC + lax.axis_index("c")

    pltpu.sync_copy(user_hbm.at[pl.ds(wid * NCH, NCH)], uidx_v)
    pltpu.sync_copy(item_hbm.at[pl.ds(wid * NCH, NCH)], iidx_v)

    def idx_body(k, carry):
        ush = k * UN
        ish = k * IN
        for j in range(NCH):
            for v in range(8):
                src = pl.ds(v * L, L)
                dst = pl.ds(k * BPW + j * 128 + v * L, L)
                thidx_v[dst] = uidx_v[j, src] + ush
                aidx_v[dst] = iidx_v[j, src] + ish
        return carry

    lax.fori_loop(0, D, idx_body, 0)

    copies = []
    for k in range(D):
        sl = pl.ds(k * BPW, BPW)
        copies.append(pltpu.async_copy(
            theta_hbm.at[thidx_v.at[sl]], th_v.at[sl], sem))
        copies.append(pltpu.async_copy(
            a_hbm.at[aidx_v.at[sl]], a_v.at[sl], sem))
    for j in range(NCH):
        sl = pl.ds(j * 128, 128)
        copies.append(pltpu.async_copy(b_hbm.at[iidx_v.at[j]], b_v.at[sl], sem))
    for c in copies:
        c.wait()

    def group_body(g, carry):
        rows = lax.iota(jnp.int32, L) + g * L
        acc = jnp.zeros((L,), jnp.float32)
        for k in range(D):
            sl = pl.ds(k * BPW + g * L, L)
            acc = acc + _softplus(a_v[sl]) * th_v[sl]
        vb = plsc.load_gather(b_v, [rows])
        res = 1.0 / (1.0 + jnp.exp(vb - acc))
        plsc.store_scatter(out_v, [rows], res)
        return carry

    lax.fori_loop(0, BPW // L, group_body, 0)
    pltpu.sync_copy(out_v, out_hbm.at[pl.ds(wid * BPW, BPW)])


def kernel(user, item, theta_table, a_table, b_table):
    u2 = user.astype(jnp.int32).reshape(NW * NCH, 128)
    i2 = item.astype(jnp.int32).reshape(NW * NCH, 128)
    th_lin, a_lin = _detile(theta_table.T, a_table.T,
                            theta_table[UN - 128:].T, a_table[IN - 128:].T)
    b1 = b_table.reshape((IN,))
    return _mirt_sc(u2, i2, th_lin, a_lin, b1)


# speedup vs baseline: 7.9668x; 1.0886x over previous
import functools

import jax
import jax.numpy as jnp
from jax import lax
from jax.experimental import pallas as pl
from jax.experimental.pallas import tpu as pltpu
from jax.experimental.pallas import tpu_sc as plsc

B = 16384
D = 16
L = 16
NC = 2
NS = 16
NW = NC * NS
BPW = B // NW
NCH = BPW // 128
UN = 1000000
IN = 100000

WW = 2048
UFULL = UN // WW
UA_C0 = UFULL * WW
UA_CW = 512
AFULL = IN // WW
AA_C0 = AFULL * WW
AA_CW = 1664
NJT_H = 512
NJA_H = 64

_LOG1P_COEF = (
    9.09903358e-08, 9.99991449e-01, -4.99801099e-01, 3.31333659e-01,
    -2.39189722e-01, 1.64781887e-01, -9.23123095e-02, 3.44179115e-02,
    -6.07475245e-03,
)


def _softplus(x):
    t = jnp.exp(-jnp.abs(x))
    p = jnp.full((L,), _LOG1P_COEF[-1], jnp.float32)
    for c in _LOG1P_COEF[-2::-1]:
        p = p * t + c
    return jnp.maximum(x, 0.0) + p


@functools.partial(
    pl.kernel,
    out_type=(jax.ShapeDtypeStruct((D * UN,), jnp.float32),
              jax.ShapeDtypeStruct((D * IN,), jnp.float32)),
    mesh=plsc.VectorSubcoreMesh(core_axis_name="c", subcore_axis_name="s"),
    compiler_params=pltpu.CompilerParams(
        needs_layout_passes=False, use_tc_tiling_on_sc=True),
    scratch_types=[
        pltpu.VMEM((8, WW), jnp.float32),
        pltpu.VMEM((8, WW), jnp.float32),
        pltpu.VMEM((8 * WW,), jnp.float32),
        pltpu.VMEM((8 * WW,), jnp.float32),
        pltpu.SemaphoreType.DMA,
        pltpu.SemaphoreType.DMA,
        pltpu.SemaphoreType.DMA,
        pltpu.SemaphoreType.DMA,
    ],
)
def _detile(th_hbm, a_hbm, tht_hbm, at_hbm, tout_hbm, aout_hbm,
            bufa_v, bufb_v, row0_v, row1_v, sem0, sem1, semr0, semr1):
    wid = lax.axis_index("s") * NC + lax.axis_index("c")

    def fire_read(src, tr, c0, cw, buf_v, semr):
        return pltpu.async_copy(
            src.at[pl.ds(tr * 8, 8), pl.ds(c0, cw)],
            buf_v.at[:, pl.ds(0, cw)], semr)

    def detile_to(src, n, tr, c0, cw, row_v, unroll=False, buf_v=None,
                  pending=None):
        if buf_v is None:
            buf_v = bufa_v
        if pending is None:
            fire_read(src, tr, c0, cw, buf_v, semr0).wait()
        else:
            pending.wait()

        def step(xi):
            for v in range(8):
                sl = pl.ds(xi * 128 + v * L, L)
                for kk in range(8):
                    dsl = pl.ds(kk * WW + xi * 128 + v * L, L)
                    row_v[dsl] = buf_v[kk, sl]

        if unroll:
            for xi in range(cw // 128):
                step(xi)
        else:
            def mv(xi, carry):
                step(xi)
                return carry

            lax.fori_loop(0, cw // 128, mv, 0)

    def fire_writes(out, n, tr, out_c0, cw, row_v, sem):
        for kk in range(8):
            pltpu.async_copy(
                row_v.at[pl.ds(kk * WW, cw)],
                out.at[pl.ds((tr * 8 + kk) * n + out_c0, cw)], sem)

    def drain_writes(out, cw, row_v, sem):
        for kk in range(8):
            pltpu.make_async_copy(
                row_v.at[pl.ds(kk * WW, cw)],
                out.at[pl.ds(kk * cw, cw)], sem).wait()

    def window(src, out, n, tr, c0, cw, out_c0=None):
        if out_c0 is None:
            out_c0 = c0
        detile_to(src, n, tr, c0, cw, row0_v)
        fire_writes(out, n, tr, out_c0, cw, row0_v, sem0)
        drain_writes(out, cw, row0_v, sem0)

    def jmap(j, half, real):
        tr = j // half
        w = lax.rem(j % half, real)
        return tr, w

    def tpair(i, carry):
        j0 = wid + (2 * i) * NW
        j1 = wid + (2 * i + 1) * NW
        tr0, w0 = jmap(j0, NJT_H, UFULL)
        tr1, w1 = jmap(j1, NJT_H, UFULL)
        r0 = fire_read(th_hbm, tr0, w0 * WW, WW, bufa_v, semr0)
        r1 = fire_read(th_hbm, tr1, w1 * WW, WW, bufb_v, semr1)

        @pl.when(i > 0)
        def _():
            drain_writes(tout_hbm, WW, row0_v, sem0)
        detile_to(th_hbm, UN, tr0, w0 * WW, WW, row0_v, unroll=True,
                  buf_v=bufa_v, pending=r0)
        fire_writes(tout_hbm, UN, tr0, w0 * WW, WW, row0_v, sem0)

        @pl.when(i > 0)
        def _():
            drain_writes(tout_hbm, WW, row1_v, sem1)
        detile_to(th_hbm, UN, tr1, w1 * WW, WW, row1_v, unroll=True,
                  buf_v=bufb_v, pending=r1)
        fire_writes(tout_hbm, UN, tr1, w1 * WW, WW, row1_v, sem1)
        return carry

    lax.fori_loop(0, NJT_H * 2 // (2 * NW), tpair, 0)
    drain_writes(tout_hbm, WW, row0_v, sem0)
    drain_writes(tout_hbm, WW, row1_v, sem1)

    def apair(i, carry):
        j0 = wid + (2 * i) * NW
        j1 = wid + (2 * i + 1) * NW
        tr0, w0 = jmap(j0, NJA_H, AFULL)
        tr1, w1 = jmap(j1, NJA_H, AFULL)
        r0 = fire_read(a_hbm, tr0, w0 * WW, WW, bufa_v, semr0)
        r1 = fire_read(a_hbm, tr1, w1 * WW, WW, bufb_v, semr1)

        @pl.when(i > 0)
        def _():
            drain_writes(aout_hbm, WW, row0_v, sem0)
        detile_to(a_hbm, IN, tr0, w0 * WW, WW, row0_v, unroll=True,
                  buf_v=bufa_v, pending=r0)
        fire_writes(aout_hbm, IN, tr0, w0 * WW, WW, row0_v, sem0)

        @pl.when(i > 0)
        def _():
            drain_writes(aout_hbm, WW, row1_v, sem1)
        detile_to(a_hbm, IN, tr1, w1 * WW, WW, row1_v, unroll=True,
                  buf_v=bufb_v, pending=r1)
        fire_writes(aout_hbm, IN, tr1, w1 * WW, WW, row1_v, sem1)
        return carry

    lax.fori_loop(0, NJA_H * 2 // (2 * NW), apair, 0)
    drain_writes(aout_hbm, WW, row0_v, sem0)
    drain_writes(aout_hbm, WW, row1_v, sem1)

    @pl.when(wid == 0)
    def _():
        window(th_hbm, tout_hbm, UN, 0, UA_C0, UA_CW)

    @pl.when(wid == 1)
    def _():
        window(th_hbm, tout_hbm, UN, 1, UA_C0, UA_CW)

    @pl.when(wid == 2)
    def _():
        window(a_hbm, aout_hbm, IN, 0, AA_C0, AA_CW)

    @pl.when(wid == 3)
    def _():
        window(a_hbm, aout_hbm, IN, 1, AA_C0, AA_CW)

    @pl.when(wid == 4)
    def _():
        window(tht_hbm, tout_hbm, UN, 0, 0, 128, out_c0=UN - 128)

    @pl.when(wid == 5)
    def _():
        window(tht_hbm, tout_hbm, UN, 1, 0, 128, out_c0=UN - 128)

    @pl.when(wid == 6)
    def _():
        window(at_hbm, aout_hbm, IN, 0, 0, 128, out_c0=IN - 128)

    @pl.when(wid == 7)
    def _():
        window(at_hbm, aout_hbm, IN, 1, 0, 128, out_c0=IN - 128)


@functools.partial(
    pl.kernel,
    out_type=jax.ShapeDtypeStruct((B,), jnp.float32),
    mesh=plsc.VectorSubcoreMesh(core_axis_name="c", subcore_axis_name="s"),
    compiler_params=pltpu.CompilerParams(
        needs_layout_passes=False, use_tc_tiling_on_sc=False),
    scratch_types=[
        pltpu.VMEM((NCH, 128), jnp.int32),
        pltpu.VMEM((NCH, 128), jnp.int32),
        pltpu.VMEM((D * BPW,), jnp.int32),
        pltpu.VMEM((D * BPW,), jnp.int32),
        pltpu.VMEM((D * BPW,), jnp.float32),
        pltpu.VMEM((D * BPW,), jnp.float32),
        pltpu.VMEM((BPW,), jnp.float32),
        pltpu.VMEM((BPW,), jnp.float32),
        pltpu.SemaphoreType.DMA,
    ],
)
def _mirt_sc(user_hbm, item_hbm, theta_hbm, a_hbm, b_hbm, out_hbm,
             uidx_v, iidx_v, thidx_v, aidx_v, th_v, a_v, b_v, out_v, sem):
    wid = lax.axis_index("s") * NC + lax.axis_index("c")

    pltpu.sync_copy(user_hbm.at[pl.ds(wid * NCH, NCH)], uidx_v)
    pltpu.sync_copy(item_hbm.at[pl.ds(wid * NCH, NCH)], iidx_v)

    def idx_body(k, carry):
        ush = k * UN
        ish = k * IN
        for j in range(NCH):
            for v in range(8):
                src = pl.ds(v * L, L)
                dst = pl.ds(k * BPW + j * 128 + v * L, L)
                thidx_v[dst] = uidx_v[j, src] + ush
                aidx_v[dst] = iidx_v[j, src] + ish
        return carry

    lax.fori_loop(0, D, idx_body, 0)

    copies = []
    for k in range(D):
        sl = pl.ds(k * BPW, BPW)
        copies.append(pltpu.async_copy(
            theta_hbm.at[thidx_v.at[sl]], th_v.at[sl], sem))
        copies.append(pltpu.async_copy(
            a_hbm.at[aidx_v.at[sl]], a_v.at[sl], sem))
    for j in range(NCH):
        sl = pl.ds(j * 128, 128)
        copies.append(pltpu.async_copy(b_hbm.at[iidx_v.at[j]], b_v.at[sl], sem))
    for c in copies:
        c.wait()

    def group_body(g, carry):
        rows = lax.iota(jnp.int32, L) + g * L
        acc = jnp.zeros((L,), jnp.float32)
        for k in range(D):
            sl = pl.ds(k * BPW + g * L, L)
            acc = acc + _softplus(a_v[sl]) * th_v[sl]
        vb = plsc.load_gather(b_v, [rows])
        res = 1.0 / (1.0 + jnp.exp(vb - acc))
        plsc.store_scatter(out_v, [rows], res)
        return carry

    lax.fori_loop(0, BPW // L, group_body, 0)
    pltpu.sync_copy(out_v, out_hbm.at[pl.ds(wid * BPW, BPW)])


def kernel(user, item, theta_table, a_table, b_table):
    u2 = user.astype(jnp.int32).reshape(NW * NCH, 128)
    i2 = item.astype(jnp.int32).reshape(NW * NCH, 128)
    th_lin, a_lin = _detile(theta_table.T, a_table.T,
                            theta_table[UN - 128:].T, a_table[IN - 128:].T)
    b1 = b_table.reshape((IN,))
    return _mirt_sc(u2, i2, th_lin, a_lin, b1)
